# baseline (speedup 1.0000x reference)
import functools

import jax
import jax.numpy as jnp
from jax import lax
from jax.experimental import pallas as pl
from jax.experimental.pallas import tpu as pltpu
from jax.experimental.pallas import tpu_sc as plsc

_B = 1024
_P = 100000
_C = 1000
_K = 32

_NC = 2
_NS = 16
_NW = _NC * _NS
_QPW = _B // _NW
_L = 16

_PMAIN = (_P // 128) * 128
_PTAIL = _P - _PMAIN
_CH = 1408
_NCH = _PMAIN // _CH
_UC = 11
_GPC = _CH // (_UC * _L)
_CAP = 256

_INF = float("inf")


_PB = 1000


def _cls_body(onehot_ref, out_ref):
    blk = onehot_ref[...]
    iota = lax.broadcasted_iota(jnp.int32, (1, _C), 1).astype(jnp.float32)
    ids = jnp.sum(blk * iota, axis=1)
    out_ref[0, 0, :] = ids.astype(jnp.int32)


_cls = pl.pallas_call(
    _cls_body,
    grid=(_P // _PB,),
    in_specs=[pl.BlockSpec((_PB, _C), lambda i: (i, 0))],
    out_specs=pl.BlockSpec((1, 1, _PB), lambda i: (i, 0, 0)),
    out_shape=jax.ShapeDtypeStruct((_P // _PB, 1, _PB), jnp.int32),
)


_mesh = plsc.VectorSubcoreMesh(core_axis_name="c", subcore_axis_name="s")


def _lex_le(ak, ai, bk, bi):
    return (ak < bk) | ((ak == bk) & (ai <= bi))


def _lex_sort16(k, i, lane):
    ks, is1 = plsc.sort_key_val(k, i)
    prev = jnp.take(ks, jnp.maximum(lane - 1, 0))
    newgrp = (ks != prev) | (lane == 0)
    rank = plsc.cumsum(newgrp.astype(jnp.int32))
    key2 = rank * 131072 + is1
    key2s, vs2 = plsc.sort_key_val(key2, ks)
    idx2 = jnp.bitwise_and(key2s, 131071)
    return vs2, idx2


def _merge16(ak, ai, bk, bi, lane):
    rbk = lax.rev(bk, (0,))
    rbi = lax.rev(bi, (0,))
    s = _lex_le(ak, ai, rbk, rbi)
    lok = jnp.where(s, ak, rbk)
    hik = jnp.where(s, rbk, ak)
    loi = jnp.where(s, ai, rbi)
    hii = jnp.where(s, rbi, ai)
    lok, loi = _lex_sort16(lok, loi, lane)
    hik, hii = _lex_sort16(hik, hii, lane)
    return (lok, loi, hik, hii)


def _lomerge32(a, b, clean, lane):
    ak0, ai0, ak1, ai1 = a
    bk0, bi0, bk1, bi1 = b
    rbk0, rbi0 = lax.rev(bk1, (0,)), lax.rev(bi1, (0,))
    rbk1, rbi1 = lax.rev(bk0, (0,)), lax.rev(bi0, (0,))
    s0 = _lex_le(ak0, ai0, rbk0, rbi0)
    s1 = _lex_le(ak1, ai1, rbk1, rbi1)
    l0 = jnp.where(s0, ak0, rbk0)
    l1 = jnp.where(s1, ak1, rbk1)
    li0 = jnp.where(s0, ai0, rbi0)
    li1 = jnp.where(s1, ai1, rbi1)
    if not clean:
        return (l0, li0, l1, li1)
    t = _lex_le(l0, li0, l1, li1)
    mk0 = jnp.where(t, l0, l1)
    mk1 = jnp.where(t, l1, l0)
    mi0 = jnp.where(t, li0, li1)
    mi1 = jnp.where(t, li1, li0)
    mk0, mi0 = _lex_sort16(mk0, mi0, lane)
    mk1, mi1 = _lex_sort16(mk1, mi1, lane)
    return (mk0, mi0, mk1, mi1)


def _make_rebuild(candv, candi, lane, roff):
    def rebuild(cnt, tau):
        del cnt, tau
        ks = [candv[pl.ds(roff + j * _L, _L)] for j in range(_CAP // _L)]
        is_ = [candi[pl.ds(roff + j * _L, _L)] for j in range(_CAP // _L)]
        pairs = [_lex_sort16(k, i, lane) for k, i in zip(ks, is_)]
        runs = [
            _merge16(pairs[2 * j][0], pairs[2 * j][1],
                     pairs[2 * j + 1][0], pairs[2 * j + 1][1], lane)
            for j in range(len(pairs) // 2)
        ]
        while len(runs) > 1:
            nxt = [
                _lomerge32(runs[2 * j], runs[2 * j + 1], True, lane)
                for j in range(len(runs) // 2)
            ]
            if len(runs) % 2:
                nxt.append(runs[-1])
            runs = nxt
        l0, li0, l1, li1 = runs[0]
        new_tau = jnp.max(l1)
        candv[pl.ds(roff, _L)] = l0
        candv[pl.ds(roff + _L, _L)] = l1
        candi[pl.ds(roff, _L)] = li0
        candi[pl.ds(roff + _L, _L)] = li1
        inf16 = jnp.full((_L,), _INF, jnp.float32)
        for j in range(2, _CAP // _L):
            candv[pl.ds(roff + j * _L, _L)] = inf16
        return jnp.int32(_K), new_tau

    return rebuild


def _noop(cnt, tau):
    return cnt, tau


@functools.partial(
    pl.kernel,
    mesh=_mesh,
    out_type=jax.ShapeDtypeStruct((_B * _C,), jnp.float32),
    scratch_types=[
        pltpu.VMEM((8, _CH), jnp.float32),
        pltpu.VMEM((8, _CH), jnp.float32),
        pltpu.VMEM((_QPW * _PTAIL,), jnp.float32),
        pltpu.VMEM((8 * _CAP,), jnp.float32),
        pltpu.VMEM((8 * _CAP,), jnp.int32),
        pltpu.VMEM((8 * _K,), jnp.int32),
        pltpu.VMEM((8 * _K,), jnp.int32),
        pltpu.VMEM((8 * _C,), jnp.float32),
        pltpu.SMEM((8,), jnp.int32),
        pltpu.SMEM((8,), jnp.float32),
        pltpu.SemaphoreType.DMA,
        pltpu.SemaphoreType.DMA,
    ],
    compiler_params=pltpu.CompilerParams(needs_layout_passes=False),
)
def _topk(dist_hbm, tail_hbm, cls_hbm, out_hbm, bufa, bufb, tails, candv,
          candi, idxg, ids_v, acc_v, cnts, taus, sema, semb):
    wid = lax.axis_index("s") * _NC + lax.axis_index("c")
    row0 = wid * _QPW
    pltpu.sync_copy(
        tail_hbm.at[pl.ds(row0 * _PTAIL, _QPW * _PTAIL)], tails
    )
    lane = lax.iota(jnp.int32, _L)
    inf16 = jnp.full((_L,), _INF, jnp.float32)

    def start(o, c, buf, sem):
        return pltpu.async_copy(
            dist_hbm.at[pl.ds(row0 + o * 8, 8), pl.ds(c * _CH, _CH)],
            buf, sem,
        )

    def wait(o, buf, sem):
        pltpu.make_async_copy(
            dist_hbm.at[pl.ds(row0 + o * 8, 8), pl.ds(0, _CH)], buf, sem
        ).wait()

    def scan_chunk(buf, c):
        def rloop(r, carry):
            roff = r * _CAP
            rebuild = _make_rebuild(candv, candi, lane, roff)

            def load_group(g):
                return [
                    buf[r, pl.ds(g * (_UC * _L) + u * _L, _L)]
                    for u in range(_UC)
                ]

            def nhit_of(vs, tau):
                tv = jnp.full((_L,), tau, jnp.float32)
                m = None
                for v in vs:
                    mu = v < tv
                    m = mu if m is None else (m | mu)
                return jnp.sum(m.astype(jnp.int32))

            def process(gprev, vs, nhit, cnt, tau):
                def hit(cnt, tau):
                    cnt, tau = lax.cond(
                        cnt > _CAP - _UC * _L, rebuild, _noop, cnt, tau
                    )
                    tv2 = jnp.full((_L,), tau, jnp.float32)
                    gbase = c * _CH + gprev * (_UC * _L)
                    for u in range(_UC):
                        mu = vs[u] < tv2
                        mi = mu.astype(jnp.int32)
                        cc = plsc.cumsum(mi)
                        pos = roff + cnt + cc - mi
                        iv = gbase + u * _L + lane
                        plsc.store_scatter(candv, [pos], vs[u], mask=mu)
                        plsc.store_scatter(candi, [pos], iv, mask=mu)
                        cnt = cnt + jnp.max(cc)
                    return cnt, tau

                return lax.cond(nhit > 0, hit, _noop, cnt, tau)

            cnt0, tau0 = cnts[r], taus[r]
            vs0 = load_group(0)
            nh0 = nhit_of(vs0, tau0)

            def gbody(g, st):
                cnt, tau, nh_prev = st[0], st[1], st[2]
                vs_prev = list(st[3:])
                vs = load_group(g)
                nh = nhit_of(vs, tau)
                cnt, tau = process(g - 1, vs_prev, nh_prev, cnt, tau)
                return (cnt, tau, nh) + tuple(vs)

            st = lax.fori_loop(1, _GPC, gbody, (cnt0, tau0, nh0) + tuple(vs0))
            cnt, tau, nh_last = st[0], st[1], st[2]
            cnt, tau = process(_GPC - 1, list(st[3:]), nh_last, cnt, tau)
            cnts[r] = cnt
            taus[r] = tau
            return carry

        lax.fori_loop(0, 8, rloop, 0)

    def octet(o, carry):
        start(o, 0, bufa, sema)

        def init(j, c2):
            candv[pl.ds(j * _L, _L)] = inf16
            return c2

        lax.fori_loop(0, (8 * _CAP) // _L, init, 0)

        def rinit(r, c2):
            cnts[r] = jnp.int32(_K)
            taus[r] = jnp.float32(_INF)
            return c2

        lax.fori_loop(0, 8, rinit, 0)

        def pair(g, c2):
            cc = 2 * g
            wait(o, bufa, sema)
            start(o, cc + 1, bufb, semb)
            scan_chunk(bufa, cc)
            wait(o, bufb, semb)
            start(o, cc + 2, bufa, sema)
            scan_chunk(bufb, cc + 1)
            return c2

        lax.fori_loop(0, (_NCH - 1) // 2, pair, 0)
        wait(o, bufa, sema)
        scan_chunk(bufa, jnp.int32(_NCH - 1))

        def rfin(r, c2):
            roff = r * _CAP
            rebuild = _make_rebuild(candv, candi, lane, roff)
            lrow = o * 8 + r
            cnt, tau = cnts[r], taus[r]
            tv = jnp.full((_L,), tau, jnp.float32)
            vs = [
                tails[pl.ds(lrow * _PTAIL + u * _L, _L)] for u in range(2)
            ]
            m = (vs[0] < tv) | (vs[1] < tv)
            nhit = jnp.sum(m.astype(jnp.int32))

            def hit(cnt, tau):
                cnt, tau = lax.cond(cnt > _CAP - 32, rebuild, _noop, cnt, tau)
                tv2 = jnp.full((_L,), tau, jnp.float32)
                for u in range(2):
                    mu = vs[u] < tv2
                    mi = mu.astype(jnp.int32)
                    cc = plsc.cumsum(mi)
                    pos = roff + cnt + cc - mi
                    iv = _PMAIN + u * _L + lane
                    plsc.store_scatter(candv, [pos], vs[u], mask=mu)
                    plsc.store_scatter(candi, [pos], iv, mask=mu)
                    cnt = cnt + jnp.max(cc)
                return cnt, tau

            cnt, tau = lax.cond(nhit > 0, hit, _noop, cnt, tau)
            cnt, tau = rebuild(cnt, tau)
            for u in range(2):
                idxg[pl.ds(r * _K + u * _L, _L)] = candi[
                    pl.ds(roff + u * _L, _L)
                ]
            return c2

        lax.fori_loop(0, 8, rfin, 0)

        gather = pltpu.async_copy(cls_hbm.at[idxg], ids_v, sema)
        zero = jnp.zeros((_L,), jnp.float32)

        def zcol(j, c2):
            acc_v[pl.ds(j * _L, _L)] = zero
            return c2

        lax.fori_loop(0, (8 * _C) // _L, zcol, 0)
        gather.wait()

        vote = jnp.full((_L,), 1.0 / _K, jnp.float32)
        lane_masks = [lane == i for i in range(_L)]

        def qvote(r, c2):
            rowbase = jnp.full((_L,), r * _C, jnp.int32)
            for half in range(_K // _L):
                ids = ids_v[pl.ds(r * _K + half * _L, _L)] + rowbase
                for i in range(_L):
                    plsc.addupdate_scatter(
                        acc_v, [ids], vote, mask=lane_masks[i]
                    )
            return c2

        lax.fori_loop(0, 8, qvote, 0)
        pltpu.sync_copy(
            acc_v, out_hbm.at[pl.ds((row0 + o * 8) * _C, 8 * _C)]
        )
        return carry

    lax.fori_loop(0, _QPW // 8, octet, 0)


def kernel(distances, one_hot_prototype_labels):
    tail = distances[:, _PMAIN:].reshape(-1)
    cls = _cls(one_hot_prototype_labels).reshape(_P)
    return _topk(distances, tail, cls).reshape(_B, _C)


# speedup vs baseline: 1.3031x; 1.3031x over previous
import functools

import jax
import jax.numpy as jnp
from jax import lax
from jax.experimental import pallas as pl
from jax.experimental.pallas import tpu as pltpu
from jax.experimental.pallas import tpu_sc as plsc

_B = 1024
_P = 100000
_C = 1000
_K = 32

_NC = 2
_NS = 16
_NW = _NC * _NS
_QPW = _B // _NW
_L = 16

_PMAIN = (_P // 128) * 128
_PTAIL = _P - _PMAIN
_CH = 1408
_NCH = _PMAIN // _CH
_UC = 11
_GPC = _CH // (_UC * _L)
_CAP = 256

_INF = float("inf")


_PB = 1000


def _cls_body(onehot_ref, out_ref):
    blk = onehot_ref[...]
    iota = lax.broadcasted_iota(jnp.int32, (1, _C), 1).astype(jnp.float32)
    ids = jnp.sum(blk * iota, axis=1)
    out_ref[0, 0, :] = ids.astype(jnp.int32)


_cls = pl.pallas_call(
    _cls_body,
    grid=(_P // _PB,),
    in_specs=[pl.BlockSpec((_PB, _C), lambda i: (i, 0))],
    out_specs=pl.BlockSpec((1, 1, _PB), lambda i: (i, 0, 0)),
    out_shape=jax.ShapeDtypeStruct((_P // _PB, 1, _PB), jnp.int32),
)


_mesh = plsc.VectorSubcoreMesh(core_axis_name="c", subcore_axis_name="s")


def _lex_le(ak, ai, bk, bi):
    return (ak < bk) | ((ak == bk) & (ai <= bi))


def _lex_sort16(k, i, lane):
    ks, is1 = plsc.sort_key_val(k, i)
    prev = jnp.take(ks, jnp.maximum(lane - 1, 0))
    newgrp = (ks != prev) | (lane == 0)
    rank = plsc.cumsum(newgrp.astype(jnp.int32))
    key2 = rank * 131072 + is1
    key2s, vs2 = plsc.sort_key_val(key2, ks)
    idx2 = jnp.bitwise_and(key2s, 131071)
    return vs2, idx2


def _merge16(ak, ai, bk, bi, lane):
    rbk = lax.rev(bk, (0,))
    rbi = lax.rev(bi, (0,))
    s = _lex_le(ak, ai, rbk, rbi)
    lok = jnp.where(s, ak, rbk)
    hik = jnp.where(s, rbk, ak)
    loi = jnp.where(s, ai, rbi)
    hii = jnp.where(s, rbi, ai)
    lok, loi = _lex_sort16(lok, loi, lane)
    hik, hii = _lex_sort16(hik, hii, lane)
    return (lok, loi, hik, hii)


def _lomerge32(a, b, clean, lane):
    ak0, ai0, ak1, ai1 = a
    bk0, bi0, bk1, bi1 = b
    rbk0, rbi0 = lax.rev(bk1, (0,)), lax.rev(bi1, (0,))
    rbk1, rbi1 = lax.rev(bk0, (0,)), lax.rev(bi0, (0,))
    s0 = _lex_le(ak0, ai0, rbk0, rbi0)
    s1 = _lex_le(ak1, ai1, rbk1, rbi1)
    l0 = jnp.where(s0, ak0, rbk0)
    l1 = jnp.where(s1, ak1, rbk1)
    li0 = jnp.where(s0, ai0, rbi0)
    li1 = jnp.where(s1, ai1, rbi1)
    if not clean:
        return (l0, li0, l1, li1)
    t = _lex_le(l0, li0, l1, li1)
    mk0 = jnp.where(t, l0, l1)
    mk1 = jnp.where(t, l1, l0)
    mi0 = jnp.where(t, li0, li1)
    mi1 = jnp.where(t, li1, li0)
    mk0, mi0 = _lex_sort16(mk0, mi0, lane)
    mk1, mi1 = _lex_sort16(mk1, mi1, lane)
    return (mk0, mi0, mk1, mi1)


def _make_rebuild(candv, candi, lane, roff):
    def rebuild(cnt, tau):
        del cnt, tau
        ks = [candv[pl.ds(roff + j * _L, _L)] for j in range(_CAP // _L)]
        is_ = [candi[pl.ds(roff + j * _L, _L)] for j in range(_CAP // _L)]
        pairs = [_lex_sort16(k, i, lane) for k, i in zip(ks, is_)]
        runs = [
            _merge16(pairs[2 * j][0], pairs[2 * j][1],
                     pairs[2 * j + 1][0], pairs[2 * j + 1][1], lane)
            for j in range(len(pairs) // 2)
        ]
        while len(runs) > 1:
            nxt = [
                _lomerge32(runs[2 * j], runs[2 * j + 1], True, lane)
                for j in range(len(runs) // 2)
            ]
            if len(runs) % 2:
                nxt.append(runs[-1])
            runs = nxt
        l0, li0, l1, li1 = runs[0]
        new_tau = jnp.max(l1)
        candv[pl.ds(roff, _L)] = l0
        candv[pl.ds(roff + _L, _L)] = l1
        candi[pl.ds(roff, _L)] = li0
        candi[pl.ds(roff + _L, _L)] = li1
        inf16 = jnp.full((_L,), _INF, jnp.float32)
        for j in range(2, _CAP // _L):
            candv[pl.ds(roff + j * _L, _L)] = inf16
        return jnp.int32(_K), new_tau

    return rebuild


def _noop(cnt, tau):
    return cnt, tau


@functools.partial(
    pl.kernel,
    mesh=_mesh,
    out_type=jax.ShapeDtypeStruct((_B * _K,), jnp.int32),
    scratch_types=[
        pltpu.VMEM((8, _CH), jnp.float32),
        pltpu.VMEM((8, _CH), jnp.float32),
        pltpu.VMEM((_QPW * _PTAIL,), jnp.float32),
        pltpu.VMEM((8 * _CAP,), jnp.float32),
        pltpu.VMEM((8 * _CAP,), jnp.int32),
        pltpu.SMEM((8,), jnp.int32),
        pltpu.SMEM((8,), jnp.float32),
        pltpu.SemaphoreType.DMA,
        pltpu.SemaphoreType.DMA,
    ],
    compiler_params=pltpu.CompilerParams(needs_layout_passes=False),
)
def _topk(dist_hbm, tail_hbm, idx_hbm, bufa, bufb, tails, candv, candi,
          cnts, taus, sema, semb):
    wid = lax.axis_index("s") * _NC + lax.axis_index("c")
    row0 = wid * _QPW
    pltpu.sync_copy(
        tail_hbm.at[pl.ds(row0 * _PTAIL, _QPW * _PTAIL)], tails
    )
    lane = lax.iota(jnp.int32, _L)
    inf16 = jnp.full((_L,), _INF, jnp.float32)

    def start(o, c, buf, sem):
        return pltpu.async_copy(
            dist_hbm.at[pl.ds(row0 + o * 8, 8), pl.ds(c * _CH, _CH)],
            buf, sem,
        )

    def wait(o, buf, sem):
        pltpu.make_async_copy(
            dist_hbm.at[pl.ds(row0 + o * 8, 8), pl.ds(0, _CH)], buf, sem
        ).wait()

    def scan_chunk(buf, c):
        def rloop(r, carry):
            roff = r * _CAP
            rebuild = _make_rebuild(candv, candi, lane, roff)

            def load_group(g):
                return [
                    buf[r, pl.ds(g * (_UC * _L) + u * _L, _L)]
                    for u in range(_UC)
                ]

            def nhit_of(vs, tau):
                tv = jnp.full((_L,), tau, jnp.float32)
                m = None
                for v in vs:
                    mu = v < tv
                    m = mu if m is None else (m | mu)
                return jnp.sum(m.astype(jnp.int32))

            def process(gprev, vs, nhit, cnt, tau):
                def hit(cnt, tau):
                    cnt, tau = lax.cond(
                        cnt > _CAP - _UC * _L, rebuild, _noop, cnt, tau
                    )
                    tv2 = jnp.full((_L,), tau, jnp.float32)
                    gbase = c * _CH + gprev * (_UC * _L)
                    for u in range(_UC):
                        mu = vs[u] < tv2
                        mi = mu.astype(jnp.int32)
                        cc = plsc.cumsum(mi)
                        pos = roff + cnt + cc - mi
                        iv = gbase + u * _L + lane
                        plsc.store_scatter(candv, [pos], vs[u], mask=mu)
                        plsc.store_scatter(candi, [pos], iv, mask=mu)
                        cnt = cnt + jnp.max(cc)
                    return cnt, tau

                return lax.cond(nhit > 0, hit, _noop, cnt, tau)

            cnt0, tau0 = cnts[r], taus[r]
            vs0 = load_group(0)
            nh0 = nhit_of(vs0, tau0)

            def gbody(g, st):
                cnt, tau, nh_prev = st[0], st[1], st[2]
                vs_prev = list(st[3:])
                vs = load_group(g)
                nh = nhit_of(vs, tau)
                cnt, tau = process(g - 1, vs_prev, nh_prev, cnt, tau)
                return (cnt, tau, nh) + tuple(vs)

            st = lax.fori_loop(1, _GPC, gbody, (cnt0, tau0, nh0) + tuple(vs0))
            cnt, tau, nh_last = st[0], st[1], st[2]
            cnt, tau = process(_GPC - 1, list(st[3:]), nh_last, cnt, tau)
            cnts[r] = cnt
            taus[r] = tau
            return carry

        lax.fori_loop(0, 8, rloop, 0)

    def octet(o, carry):
        start(o, 0, bufa, sema)

        def init(j, c2):
            candv[pl.ds(j * _L, _L)] = inf16
            return c2

        lax.fori_loop(0, (8 * _CAP) // _L, init, 0)

        def rinit(r, c2):
            cnts[r] = jnp.int32(_K)
            taus[r] = jnp.float32(_INF)
            return c2

        lax.fori_loop(0, 8, rinit, 0)

        def pair(g, c2):
            cc = 2 * g
            wait(o, bufa, sema)
            start(o, cc + 1, bufb, semb)
            scan_chunk(bufa, cc)
            wait(o, bufb, semb)
            start(o, cc + 2, bufa, sema)
            scan_chunk(bufb, cc + 1)
            return c2

        lax.fori_loop(0, (_NCH - 1) // 2, pair, 0)
        wait(o, bufa, sema)
        scan_chunk(bufa, jnp.int32(_NCH - 1))

        def rfin(r, c2):
            roff = r * _CAP
            rebuild = _make_rebuild(candv, candi, lane, roff)
            lrow = o * 8 + r
            cnt, tau = cnts[r], taus[r]
            tv = jnp.full((_L,), tau, jnp.float32)
            vs = [
                tails[pl.ds(lrow * _PTAIL + u * _L, _L)] for u in range(2)
            ]
            m = (vs[0] < tv) | (vs[1] < tv)
            nhit = jnp.sum(m.astype(jnp.int32))

            def hit(cnt, tau):
                cnt, tau = lax.cond(cnt > _CAP - 32, rebuild, _noop, cnt, tau)
                tv2 = jnp.full((_L,), tau, jnp.float32)
                for u in range(2):
                    mu = vs[u] < tv2
                    mi = mu.astype(jnp.int32)
                    cc = plsc.cumsum(mi)
                    pos = roff + cnt + cc - mi
                    iv = _PMAIN + u * _L + lane
                    plsc.store_scatter(candv, [pos], vs[u], mask=mu)
                    plsc.store_scatter(candi, [pos], iv, mask=mu)
                    cnt = cnt + jnp.max(cc)
                return cnt, tau

            cnt, tau = lax.cond(nhit > 0, hit, _noop, cnt, tau)
            cnt, tau = rebuild(cnt, tau)
            pltpu.sync_copy(
                candi.at[pl.ds(roff, _K)],
                idx_hbm.at[pl.ds((row0 + lrow) * _K, _K)],
            )
            return c2

        lax.fori_loop(0, 8, rfin, 0)
        return carry

    lax.fori_loop(0, _QPW // 8, octet, 0)


@functools.partial(
    pl.kernel,
    mesh=_mesh,
    out_type=jax.ShapeDtypeStruct((_B * _C,), jnp.float32),
    scratch_types=[
        pltpu.VMEM((_QPW * _K,), jnp.int32),
        pltpu.VMEM((_QPW * _K,), jnp.int32),
        pltpu.VMEM((_QPW * _C,), jnp.float32),
        pltpu.SemaphoreType.DMA,
    ],
    compiler_params=pltpu.CompilerParams(needs_layout_passes=False),
)
def _vote(idx_hbm, cls_hbm, out_hbm, idx_v, ids_v, acc_v, sem):
    wid = lax.axis_index("s") * _NC + lax.axis_index("c")
    base = wid * (_QPW * _K)
    pltpu.sync_copy(idx_hbm.at[pl.ds(base, _QPW * _K)], idx_v)
    gather = pltpu.async_copy(cls_hbm.at[idx_v], ids_v, sem)

    zero = jnp.zeros((_L,), jnp.float32)

    def zcol(j, carry):
        acc_v[pl.ds(j * _L, _L)] = zero
        return carry

    lax.fori_loop(0, (_QPW * _C) // _L, zcol, 0)
    gather.wait()

    vote = jnp.full((_L,), 1.0 / _K, jnp.float32)
    lane = lax.iota(jnp.int32, _L)
    lane_masks = [lane == i for i in range(_L)]

    def qbody(q, carry):
        rowbase = jnp.full((_L,), q * _C, jnp.int32)
        for half in range(_K // _L):
            ids = ids_v[pl.ds(q * _K + half * _L, _L)] + rowbase
            for i in range(_L):
                plsc.addupdate_scatter(acc_v, [ids], vote, mask=lane_masks[i])
        return carry

    lax.fori_loop(0, _QPW, qbody, 0)
    pltpu.sync_copy(acc_v, out_hbm.at[pl.ds(wid * (_QPW * _C), _QPW * _C)])


def kernel(distances, one_hot_prototype_labels):
    tail = distances[:, _PMAIN:].reshape(-1)
    cls = _cls(one_hot_prototype_labels).reshape(_P)
    idx = _topk(distances, tail)
    return _vote(idx, cls).reshape(_B, _C)
